# baseline (speedup 1.0000x reference)
import functools

import jax
import jax.numpy as jnp
from jax import lax
from jax.experimental import pallas as pl
from jax.experimental.pallas import tpu as pltpu
from jax.experimental.pallas import tpu_sc as plsc

B = 4096
L = 50
D_O = 64
VOCAB = 1000000
NC, NS = 2, 16
NW = NC * NS
B_PER_W = B // NW
NB = 8
N_CHUNKS = B_PER_W // NB

_mesh = plsc.VectorSubcoreMesh(core_axis_name="c", subcore_axis_name="s")


@functools.partial(
    pl.kernel,
    mesh=_mesh,
    out_type=jax.ShapeDtypeStruct((B, L, D_O), jnp.float32),
    scratch_types=[
        pltpu.VMEM((NB, L), jnp.int32),
        pltpu.VMEM((NB, 16 * L), jnp.float32),
        pltpu.VMEM((NB, L, D_O), jnp.float32),
        pltpu.SemaphoreType.DMA,
    ],
    compiler_params=pltpu.CompilerParams(use_tc_tiling_on_sc=False),
)
def _emb_kernel(x_hbm, coords_hbm, table_hbm, out_hbm, xv, cv, obuf, sem):
    wid = lax.axis_index("s") * NC + lax.axis_index("c")
    b0 = wid * B_PER_W

    lane = lax.iota(jnp.int32, 16)
    is_emb = lane < 12

    def chunk_body(i, carry):
        b = b0 + i * NB
        pltpu.sync_copy(x_hbm.at[pl.ds(b, NB)], xv)
        copies = [
            pltpu.async_copy(table_hbm.at[xv.at[j]], obuf.at[j], sem)
            for j in range(NB)
        ]
        pltpu.sync_copy(coords_hbm.at[pl.ds(b, NB)], cv)
        for c in copies:
            c.wait()

        for j in range(NB):
            def row_body(l, c2):
                v = obuf[j, l, pl.ds(48, 16)]
                cvec = cv[j, pl.ds(16 * l, 16)]
                obuf[j, l, pl.ds(48, 16)] = jnp.where(is_emb, v, cvec)
                return c2
            lax.fori_loop(0, L, row_body, 0)

        pltpu.sync_copy(obuf, out_hbm.at[pl.ds(b, NB)])
        return carry

    lax.fori_loop(0, N_CHUNKS, chunk_body, 0)


_TR = 38400
_NBLK = (VOCAB + _TR - 1) // _TR
_VPAD = _NBLK * _TR


def _tp_body(t_ref, o_ref):
    blk = t_ref[...]
    rows = jnp.transpose(blk, (1, 0))
    lo = jnp.pad(rows[: _TR // 2], ((0, 0), (0, 4)))
    hi = jnp.pad(rows[_TR // 2 :], ((0, 0), (0, 4)))
    o_ref[...] = jnp.concatenate([lo, hi], axis=1)


def _transpose_pad(table_t):
    return pl.pallas_call(
        _tp_body,
        grid=(_NBLK,),
        in_specs=[pl.BlockSpec((60, _TR), lambda i: (0, i))],
        out_specs=pl.BlockSpec((_TR // 2, 2 * D_O), lambda i: (i, 0)),
        out_shape=jax.ShapeDtypeStruct((_VPAD // 2, 2 * D_O), jnp.float32),
    )(table_t)


def kernel(x, coordinates, table):
    tpad = _transpose_pad(table.T).reshape(_VPAD, D_O)
    cpad = jnp.pad(coordinates, ((0, 0), (0, 0), (12, 0))).reshape(B, 16 * L)
    xi = x.astype(jnp.int32)
    blk = xi // _TR
    w = xi - blk * _TR
    p = (w >= _TR // 2).astype(jnp.int32)
    xf = blk * _TR + 2 * (w - (_TR // 2) * p) + p
    return _emb_kernel(xf, cpad, tpad)


# speedup vs baseline: 1.0074x; 1.0074x over previous
import functools

import jax
import jax.numpy as jnp
from jax import lax
from jax.experimental import pallas as pl
from jax.experimental.pallas import tpu as pltpu
from jax.experimental.pallas import tpu_sc as plsc

B = 4096
L = 50
D_O = 64
VOCAB = 1000000
NC, NS = 2, 16
NW = NC * NS
B_PER_W = B // NW
NB = 8
N_CHUNKS = B_PER_W // NB

_mesh = plsc.VectorSubcoreMesh(core_axis_name="c", subcore_axis_name="s")


@functools.partial(
    pl.kernel,
    mesh=_mesh,
    out_type=jax.ShapeDtypeStruct((B, L, D_O), jnp.float32),
    scratch_types=[
        pltpu.VMEM((NB, L), jnp.int32),
        pltpu.VMEM((NB, 16 * L), jnp.float32),
        pltpu.VMEM((NB, L, D_O), jnp.float32),
        pltpu.SemaphoreType.DMA,
    ],
    compiler_params=pltpu.CompilerParams(use_tc_tiling_on_sc=False),
)
def _emb_kernel(x_hbm, coords_hbm, table_hbm, out_hbm, xv, cv, obuf, sem):
    wid = lax.axis_index("s") * NC + lax.axis_index("c")
    b0 = wid * B_PER_W

    lane = lax.iota(jnp.int32, 16)
    is_emb = lane < 12

    def chunk_body(i, carry):
        b = b0 + i * NB
        pltpu.sync_copy(x_hbm.at[pl.ds(b, NB)], xv)
        copies = [
            pltpu.async_copy(table_hbm.at[xv.at[j]], obuf.at[j], sem)
            for j in range(NB)
        ]
        pltpu.sync_copy(coords_hbm.at[pl.ds(b, NB)], cv)
        for c in copies:
            c.wait()

        for j in range(NB):
            def row_body(l, c2):
                v = obuf[j, l, pl.ds(48, 16)]
                cvec = cv[j, pl.ds(16 * l, 16)]
                obuf[j, l, pl.ds(48, 16)] = jnp.where(is_emb, v, cvec)
                return c2
            lax.fori_loop(0, L, row_body, 0)

        pltpu.sync_copy(obuf, out_hbm.at[pl.ds(b, NB)])
        return carry

    lax.fori_loop(0, N_CHUNKS, chunk_body, 0)


_TR = 25600
_NBLK = (VOCAB + _TR - 1) // _TR
_VPAD = _NBLK * _TR


def _tp_body(t_ref, o_ref):
    blk = t_ref[...]
    rows = jnp.transpose(blk, (1, 0))
    lo = jnp.pad(rows[: _TR // 2], ((0, 0), (0, 4)))
    hi = jnp.pad(rows[_TR // 2 :], ((0, 0), (0, 4)))
    o_ref[...] = jnp.concatenate([lo, hi], axis=1)


def _transpose_pad(table_t):
    return pl.pallas_call(
        _tp_body,
        grid=(_NBLK,),
        in_specs=[pl.BlockSpec((60, _TR), lambda i: (0, i))],
        out_specs=pl.BlockSpec((_TR // 2, 2 * D_O), lambda i: (i, 0)),
        out_shape=jax.ShapeDtypeStruct((_VPAD // 2, 2 * D_O), jnp.float32),
    )(table_t)


def kernel(x, coordinates, table):
    tpad = _transpose_pad(table.T).reshape(_VPAD, D_O)
    cpad = jnp.pad(coordinates, ((0, 0), (0, 0), (12, 0))).reshape(B, 16 * L)
    xi = x.astype(jnp.int32)
    blk = xi // _TR
    w = xi - blk * _TR
    p = (w >= _TR // 2).astype(jnp.int32)
    xf = blk * _TR + 2 * (w - (_TR // 2) * p) + p
    return _emb_kernel(xf, cpad, tpad)


# speedup vs baseline: 1.0527x; 1.0450x over previous
import functools

import jax
import jax.numpy as jnp
from jax import lax
from jax.experimental import pallas as pl
from jax.experimental.pallas import tpu as pltpu
from jax.experimental.pallas import tpu_sc as plsc

B = 4096
L = 50
D_O = 64
VOCAB = 1000000
NC, NS = 2, 16
NW = NC * NS
B_PER_W = B // NW
NB = 8
N_CHUNKS = B_PER_W // NB

_mesh = plsc.VectorSubcoreMesh(core_axis_name="c", subcore_axis_name="s")


@functools.partial(
    pl.kernel,
    mesh=_mesh,
    out_type=jax.ShapeDtypeStruct((B, L, D_O), jnp.float32),
    scratch_types=[
        pltpu.VMEM((NB, L), jnp.int32),
        pltpu.VMEM((NB, L), jnp.int32),
        pltpu.VMEM((NB, 16 * L), jnp.float32),
        pltpu.VMEM((NB, 16 * L), jnp.float32),
        pltpu.VMEM((NB, L, D_O), jnp.float32),
        pltpu.VMEM((NB, L, D_O), jnp.float32),
        pltpu.SemaphoreType.DMA,
        pltpu.SemaphoreType.DMA,
        pltpu.SemaphoreType.DMA,
        pltpu.SemaphoreType.DMA,
    ],
    compiler_params=pltpu.CompilerParams(use_tc_tiling_on_sc=False),
)
def _emb_kernel(x_hbm, coords_hbm, table_hbm, out_hbm,
                xv_a, xv_b, cv_a, cv_b, ob_a, ob_b,
                gs_a, gs_b, ws_a, ws_b):
    wid = lax.axis_index("s") * NC + lax.axis_index("c")
    b0 = wid * B_PER_W

    lane = lax.iota(jnp.int32, 16)
    is_emb = lane < 12
    bufs = [(xv_a, cv_a, ob_a, gs_a, ws_a), (xv_b, cv_b, ob_b, gs_b, ws_b)]

    def load_inputs(k, xv, cv):
        b = b0 + k * NB
        pltpu.sync_copy(x_hbm.at[pl.ds(b, NB)], xv)
        pltpu.sync_copy(coords_hbm.at[pl.ds(b, NB)], cv)

    def fire_gathers(xv, ob, gs):
        return [
            pltpu.async_copy(table_hbm.at[xv.at[j]], ob.at[j], gs)
            for j in range(NB)
        ]

    def patch(ob, cv):
        for j in range(NB):
            def row_body(l, c2):
                v = ob[j, l, pl.ds(48, 16)]
                cvec = cv[j, pl.ds(16 * l, 16)]
                ob[j, l, pl.ds(48, 16)] = jnp.where(is_emb, v, cvec)
                return c2
            lax.fori_loop(0, L, row_body, 0)

    load_inputs(0, xv_a, cv_a)
    pend_g = {0: fire_gathers(xv_a, ob_a, gs_a)}
    pend_w = {0: None, 1: None}
    for k in range(N_CHUNKS):
        p = k % 2
        xv, cv, ob, gs, ws = bufs[p]
        if k + 1 < N_CHUNKS:
            q = 1 - p
            xq, cq, oq, gq, _ = bufs[q]
            load_inputs(k + 1, xq, cq)
        for d in pend_g.pop(p):
            d.wait()
        if k + 1 < N_CHUNKS:
            if pend_w[q] is not None:
                pend_w[q].wait()
                pend_w[q] = None
            pend_g[q] = fire_gathers(xq, oq, gq)
        patch(ob, cv)
        pend_w[p] = pltpu.async_copy(ob, out_hbm.at[pl.ds(b0 + k * NB, NB)], ws)
    for p in (0, 1):
        if pend_w[p] is not None:
            pend_w[p].wait()


_TR = 25600
_NBLK = (VOCAB + _TR - 1) // _TR
_VPAD = _NBLK * _TR


def _tp_body(t_ref, o_ref):
    blk = t_ref[...]
    rows = jnp.transpose(blk, (1, 0))
    lo = jnp.pad(rows[: _TR // 2], ((0, 0), (0, 4)))
    hi = jnp.pad(rows[_TR // 2 :], ((0, 0), (0, 4)))
    o_ref[...] = jnp.concatenate([lo, hi], axis=1)


def _transpose_pad(table_t):
    return pl.pallas_call(
        _tp_body,
        grid=(_NBLK,),
        in_specs=[pl.BlockSpec((60, _TR), lambda i: (0, i))],
        out_specs=pl.BlockSpec((_TR // 2, 2 * D_O), lambda i: (i, 0)),
        out_shape=jax.ShapeDtypeStruct((_VPAD // 2, 2 * D_O), jnp.float32),
    )(table_t)


def kernel(x, coordinates, table):
    tpad = _transpose_pad(table.T).reshape(_VPAD, D_O)
    cpad = jnp.pad(coordinates, ((0, 0), (0, 0), (12, 0))).reshape(B, 16 * L)
    xi = x.astype(jnp.int32)
    blk = xi // _TR
    w = xi - blk * _TR
    p = (w >= _TR // 2).astype(jnp.int32)
    xf = blk * _TR + 2 * (w - (_TR // 2) * p) + p
    return _emb_kernel(xf, cpad, tpad)
